# baseline (speedup 1.0000x reference)
import functools

import jax
import jax.numpy as jnp
from jax import lax
from jax.experimental import pallas as pl
from jax.experimental.pallas import tpu as pltpu
from jax.experimental.pallas import tpu_sc as plsc

EMBED_DIM = 64
NUM_WORKERS = 32
CHUNK = 128


def _cdiv(a, b):
    return (a + b - 1) // b


@functools.partial(jax.jit, static_argnames=("chunks_per_worker",))
def _embed_gather(idx, table, *, chunks_per_worker):
    n_rows_pad = idx.shape[0]
    per_w = chunks_per_worker * CHUNK
    mesh = plsc.VectorSubcoreMesh(core_axis_name="c", subcore_axis_name="s")

    @functools.partial(
        pl.kernel,
        mesh=mesh,
        compiler_params=pltpu.CompilerParams(use_tc_tiling_on_sc=False),
        out_type=jax.ShapeDtypeStruct((n_rows_pad, EMBED_DIM), jnp.float32),
        scratch_types=[
            pltpu.VMEM((per_w,), jnp.int32),
            pltpu.VMEM((CHUNK, EMBED_DIM), jnp.float32),
            pltpu.SemaphoreType.DMA,
        ],
    )
    def k(idx_hbm, table_hbm, out_hbm, idx_v, rows_v, sem):
        wid = lax.axis_index("s") * 2 + lax.axis_index("c")
        base = wid * per_w
        pltpu.sync_copy(idx_hbm.at[pl.ds(base, per_w)], idx_v)

        def body(j, carry):
            pltpu.async_copy(
                table_hbm.at[idx_v.at[pl.ds(j * CHUNK, CHUNK)]], rows_v, sem
            ).wait()
            pltpu.sync_copy(
                rows_v, out_hbm.at[pl.ds(base + j * CHUNK, CHUNK)]
            )
            return carry

        lax.fori_loop(0, chunks_per_worker, body, 0, unroll=False)

    return k(idx, table)


def kernel(tokens, emb_weight):
    n = tokens.shape[0]
    cpw = _cdiv(n, NUM_WORKERS * CHUNK)
    n_pad = cpw * NUM_WORKERS * CHUNK
    idx = jnp.pad(tokens.astype(jnp.int32), (0, n_pad - n))
    out = _embed_gather(idx, emb_weight, chunks_per_worker=cpw)
    return out[:n]


# speedup vs baseline: 1.1076x; 1.1076x over previous
import functools

import jax
import jax.numpy as jnp
from jax import lax
from jax.experimental import pallas as pl
from jax.experimental.pallas import tpu as pltpu
from jax.experimental.pallas import tpu_sc as plsc

EMBED_DIM = 64
NUM_WORKERS = 32
CHUNK = 112
NBUF = 7


def _cdiv(a, b):
    return (a + b - 1) // b


@functools.partial(jax.jit, static_argnames=("chunks_per_worker",))
def _embed_gather(idx, table, *, chunks_per_worker):
    n_rows_pad = idx.shape[0]
    per_w = chunks_per_worker * CHUNK
    groups = chunks_per_worker // NBUF
    mesh = plsc.VectorSubcoreMesh(core_axis_name="c", subcore_axis_name="s")

    @functools.partial(
        pl.kernel,
        mesh=mesh,
        compiler_params=pltpu.CompilerParams(use_tc_tiling_on_sc=False),
        out_type=jax.ShapeDtypeStruct((n_rows_pad, EMBED_DIM), jnp.float32),
        scratch_types=[
            pltpu.VMEM((per_w,), jnp.int32),
            pltpu.VMEM((NBUF, CHUNK, EMBED_DIM), jnp.float32),
            pltpu.SemaphoreType.DMA((NBUF,)),
            pltpu.SemaphoreType.DMA((NBUF,)),
        ],
    )
    def k(idx_hbm, table_hbm, out_hbm, idx_v, rows_v, gsem, osem):
        wid = lax.axis_index("s") * 2 + lax.axis_index("c")
        base = wid * per_w
        pltpu.sync_copy(idx_hbm.at[pl.ds(base, per_w)], idx_v)

        def gather_start(j, b):
            pltpu.async_copy(
                table_hbm.at[idx_v.at[pl.ds(j * CHUNK, CHUNK)]],
                rows_v.at[b],
                gsem.at[b],
            )

        def gather_wait(b):
            pltpu.make_async_copy(
                table_hbm.at[pl.ds(0, CHUNK)], rows_v.at[b], gsem.at[b]
            ).wait()

        def out_start(j, b):
            pltpu.async_copy(
                rows_v.at[b],
                out_hbm.at[pl.ds(base + j * CHUNK, CHUNK)],
                osem.at[b],
            )

        def out_wait(b):
            pltpu.make_async_copy(
                rows_v.at[b], out_hbm.at[pl.ds(base, CHUNK)], osem.at[b]
            ).wait()

        for b in range(NBUF):
            gather_start(b, b)

        @pl.loop(0, groups - 1)
        def _(g):
            for b in range(NBUF):
                j = g * NBUF + b
                gather_wait(b)
                out_start(j, b)
                out_wait(b)
                gather_start(j + NBUF, b)

        for b in range(NBUF):
            j = (groups - 1) * NBUF + b
            gather_wait(b)
            out_start(j, b)
        for b in range(NBUF):
            out_wait(b)

    return k(idx, table)


def kernel(tokens, emb_weight):
    n = tokens.shape[0]
    cpw = _cdiv(_cdiv(n, NUM_WORKERS * CHUNK), NBUF) * NBUF
    n_pad = cpw * NUM_WORKERS * CHUNK
    idx = jnp.pad(tokens.astype(jnp.int32), (0, n_pad - n))
    out = _embed_gather(idx, emb_weight, chunks_per_worker=cpw)
    return out[:n]
